# baseline (speedup 1.0000x reference)
import functools

import jax
import jax.numpy as jnp
from jax import lax
from jax.experimental import pallas as pl
from jax.experimental.pallas import tpu as pltpu
from jax.experimental.pallas import tpu_sc as plsc

N = 10000
M = 10000
E = 320000
D = 128
H = D // 2

NC = 2
NS = 16
NW = NC * NS
B = 125
NBUF = 4
BC = 80
NCH_C = E // (NW * BC)
NCH_S = E // (NS * B)
MP = 10240
RPT = MP // NS
ZCH = RPT // 5

_MESH = dict(core_axis_name="c", subcore_axis_name="s")


def _sc_count_body(vidx_h, eidx_h, dv_out, de_out,
                   vidx_v, eidx_v, ones_v, zbuf, acc):
    c = lax.axis_index("c")
    s = lax.axis_index("s")
    wid = c * NS + s
    pltpu.sync_copy(vidx_h.at[wid], vidx_v)
    pltpu.sync_copy(eidx_h.at[wid], eidx_v)

    ones16 = jnp.ones((16,), jnp.float32)
    zero16 = jnp.zeros((16,), jnp.float32)

    @pl.loop(0, BC // 16)
    def _(i):
        ones_v[pl.ds(i * 16, 16)] = ones16

    @pl.loop(0, RPT // 16)
    def _(i):
        zbuf[pl.ds(i * 16, 16)] = zero16

    for idx_v, out in ((vidx_v, dv_out), (eidx_v, de_out)):
        pltpu.sync_copy(zbuf, acc.at[pl.ds(s * RPT, RPT)])
        plsc.subcore_barrier()

        @pl.loop(0, NCH_C)
        def _(j, idx_v=idx_v):
            pltpu.sync_copy(ones_v, acc.at[idx_v.at[j]], add=True)

        plsc.subcore_barrier()
        pltpu.sync_copy(acc.at[pl.ds(s * RPT, RPT)],
                        out.at[c, pl.ds(s * RPT, RPT)])
        plsc.subcore_barrier()


@functools.partial(
    pl.kernel,
    out_type=(jax.ShapeDtypeStruct((NC, MP), jnp.float32),
              jax.ShapeDtypeStruct((NC, MP), jnp.float32)),
    mesh=plsc.VectorSubcoreMesh(**_MESH),
    scratch_types=[
        pltpu.VMEM((NCH_C, BC), jnp.int32),
        pltpu.VMEM((NCH_C, BC), jnp.int32),
        pltpu.VMEM((BC,), jnp.float32),
        pltpu.VMEM((RPT,), jnp.float32),
        pltpu.VMEM_SHARED((MP,), jnp.float32),
    ],
    compiler_params=pltpu.CompilerParams(use_tc_tiling_on_sc=False),
)
def _sc_counts(vidx_h, eidx_h, dv_out, de_out,
               vidx_v, eidx_v, ones_v, zbuf, acc):
    _sc_count_body(vidx_h, eidx_h, dv_out, de_out,
                   vidx_v, eidx_v, ones_v, zbuf, acc)


def _sc_seg_body(table_h, gidx_h, sidx_h, deg_h, ph_h, md_h, out,
                 gidx_v, sidx_v, rows, zbuf, dg_v, sc_v, pm_v, gsems, ssems,
                 acc):
    c = lax.axis_index("c")
    s = lax.axis_index("s")
    pltpu.sync_copy(ph_h, pm_v.at[0])
    pltpu.sync_copy(md_h, pm_v.at[1])
    ph = lax.reduce_max(pm_v[0], axes=(0,))
    pltpu.sync_copy(gidx_h.at[ph, c, s], gidx_v)
    pltpu.sync_copy(sidx_h.at[ph, s], sidx_v)
    pltpu.sync_copy(deg_h.at[0, pl.ds(s * RPT, RPT)], dg_v.at[0])
    pltpu.sync_copy(deg_h.at[1, pl.ds(s * RPT, RPT)], dg_v.at[1])

    zero16 = jnp.zeros((16,), jnp.float32)
    one16 = jnp.ones((16,), jnp.float32)

    @pl.loop(0, ZCH)
    def _(i):
        for k in range(H // 16):
            zbuf[i, pl.ds(16 * k, 16)] = zero16

    md = pm_v[1]

    @pl.loop(0, RPT // 16)
    def _(i):
        d = dg_v[0, pl.ds(i * 16, 16)] + dg_v[1, pl.ds(i * 16, 16)]
        dei = jnp.where(d > 0, 1.0 / jnp.where(d > 0, d, 1.0), 0.0)
        sc_v[pl.ds(i * 16, 16)] = jnp.where(md > 0, dei, one16)

    for k in range(5):
        pltpu.sync_copy(zbuf, acc.at[pl.ds(s * RPT + k * ZCH, ZCH)])
    plsc.subcore_barrier()

    def _gather(j, b):
        return pltpu.make_async_copy(table_h.at[gidx_v.at[j]],
                                     rows.at[b], gsems[b])

    def _scatter(j, b):
        return pltpu.make_async_copy(rows.at[b], acc.at[sidx_v.at[j]],
                                     ssems[b])

    for k in range(NBUF - 1):
        _gather(k, k).start()

    @pl.loop(0, NCH_S, step=NBUF)
    def _(j):
        for b in range(NBUF):
            jj = j + b
            nxt = jj + NBUF - 1
            bn = (b + NBUF - 1) % NBUF

            _gather(jj, b).wait()
            pltpu.async_copy(rows.at[b], acc.at[sidx_v.at[jj]],
                             ssems[b], add=True)

            def _refill(jj=jj, bn=bn, nxt=nxt):
                _scatter(jj - 1, bn).wait()

                @pl.when(nxt < NCH_S)
                def _():
                    _gather(nxt, bn).start()

            if b == 0:
                @pl.when(j > 0)
                def _():
                    _refill()
                @pl.when(j == 0)
                def _():
                    @pl.when(NBUF - 1 < NCH_S)
                    def _():
                        _gather(NBUF - 1, NBUF - 1).start()
            else:
                _refill()

    _scatter(NCH_S - 1, (NCH_S - 1) % NBUF).wait()
    plsc.subcore_barrier()

    md_s = lax.reduce_max(pm_v[1], axes=(0,))

    @pl.when(md_s > 0)
    def _():
        for k in range(5):
            base = s * RPT + k * ZCH
            pltpu.sync_copy(acc.at[pl.ds(base, ZCH)], zbuf)

            @pl.loop(0, ZCH // 16)
            def _(g, k=k):
                scvec = sc_v[pl.ds(k * ZCH + g * 16, 16)]
                for rr in range(16):
                    r = g * 16 + rr
                    for kk in range(H // 16):
                        zbuf[r, pl.ds(kk * 16, 16)] = (
                            zbuf[r, pl.ds(kk * 16, 16)] * scvec[rr])

            pltpu.sync_copy(zbuf, out.at[c, pl.ds(base, ZCH)])

    @pl.when(md_s == 0)
    def _():
        pltpu.sync_copy(acc.at[pl.ds(s * RPT, RPT)],
                        out.at[c, pl.ds(s * RPT, RPT)])


@functools.partial(
    pl.kernel,
    out_type=jax.ShapeDtypeStruct((NC, MP, H), jnp.float32),
    mesh=plsc.VectorSubcoreMesh(**_MESH),
    scratch_types=[
        pltpu.VMEM((NCH_S, B), jnp.int32),
        pltpu.VMEM((NCH_S, B), jnp.int32),
        pltpu.VMEM((NBUF, B, H), jnp.float32),
        pltpu.VMEM((ZCH, H), jnp.float32),
        pltpu.VMEM((NC, RPT), jnp.float32),
        pltpu.VMEM((RPT,), jnp.float32),
        pltpu.VMEM((2, 16), jnp.int32),
        [pltpu.SemaphoreType.DMA] * NBUF,
        [pltpu.SemaphoreType.DMA] * NBUF,
        pltpu.VMEM_SHARED((MP, H), jnp.float32),
    ],
    compiler_params=pltpu.CompilerParams(use_tc_tiling_on_sc=False,
                                         needs_layout_passes=False),
)
def _sc_seg(table_h, gidx_h, sidx_h, deg_h, ph_h, md_h, out,
            gidx_v, sidx_v, rows, zbuf, dg_v, sc_v, pm_v, gsems, ssems, acc):
    _sc_seg_body(table_h, gidx_h, sidx_h, deg_h, ph_h, md_h, out,
                 gidx_v, sidx_v, rows, zbuf, dg_v, sc_v, pm_v, gsems, ssems,
                 acc)


R = 400
GRID = N // R


def _dvis_of(dvp_blk):
    dv = dvp_blk[:, 0] + dvp_blk[:, 1]
    return jnp.where(dv > 0, lax.rsqrt(jnp.where(dv > 0, dv, 1.0)), 0.0)


def _half_select(full, cid):
    return jnp.where(cid == 0, full[:, :H], full[:, H:])


def _tc_lin1_body(x_ref, w_ref, b_ref, dvp_ref, o_ref):
    cid = pl.program_id(1)
    h = lax.dot_general(x_ref[...], w_ref[...],
                        (((1,), (1,)), ((), ())),
                        preferred_element_type=jnp.float32)
    h = (h + b_ref[...]) * _dvis_of(dvp_ref[...])[:, None]
    o_ref[0] = _half_select(h, cid)


def _tc_lin1(x, W1, b1, dvp):
    return pl.pallas_call(
        _tc_lin1_body,
        grid=(GRID, NC),
        in_specs=[
            pl.BlockSpec((R, D), lambda i, c: (i, 0)),
            pl.BlockSpec((D, D), lambda i, c: (0, 0)),
            pl.BlockSpec((1, D), lambda i, c: (0, 0)),
            pl.BlockSpec((R, NC), lambda i, c: (i, 0)),
        ],
        out_specs=pl.BlockSpec((1, R, H), lambda i, c: (c, i, 0)),
        out_shape=jax.ShapeDtypeStruct((NC, MP, H), jnp.float32),
    )(x, W1, b1, dvp)


def _tc_lin2_body(xo_ref, dvp_ref, w_ref, b_ref, o_ref):
    cid = pl.program_id(1)
    dvis = _dvis_of(dvp_ref[...])
    t = jnp.concatenate([xo_ref[0], xo_ref[1]], axis=1)
    t = jax.nn.relu(t * dvis[:, None])
    h = lax.dot_general(t, w_ref[...], (((1,), (1,)), ((), ())),
                        preferred_element_type=jnp.float32)
    h = (h + b_ref[...]) * dvis[:, None]
    o_ref[0] = _half_select(h, cid)


def _tc_lin2(xo, dvp, W2, b2):
    return pl.pallas_call(
        _tc_lin2_body,
        grid=(GRID, NC),
        in_specs=[
            pl.BlockSpec((NC, R, H), lambda i, c: (0, i, 0)),
            pl.BlockSpec((R, NC), lambda i, c: (i, 0)),
            pl.BlockSpec((D, D), lambda i, c: (0, 0)),
            pl.BlockSpec((1, D), lambda i, c: (0, 0)),
        ],
        out_specs=pl.BlockSpec((1, R, H), lambda i, c: (c, i, 0)),
        out_shape=jax.ShapeDtypeStruct((NC, MP, H), jnp.float32),
    )(xo, dvp, W2, b2)


def _tc_fin_body(xo_ref, dvp_ref, o_ref):
    dvis = _dvis_of(dvp_ref[...])[:, None]
    o_ref[...] = jax.nn.relu(
        jnp.concatenate([xo_ref[0], xo_ref[1]], axis=1) * dvis)


def _tc_fin(xo, dvp):
    return pl.pallas_call(
        _tc_fin_body,
        grid=(GRID,),
        in_specs=[
            pl.BlockSpec((NC, R, H), lambda i: (0, i, 0)),
            pl.BlockSpec((R, NC), lambda i: (i, 0)),
        ],
        out_specs=pl.BlockSpec((R, D), lambda i: (i, 0)),
        out_shape=jax.ShapeDtypeStruct((N, D), jnp.float32),
    )(xo, dvp)


def kernel(x, hyperedge_index, W1, b1, W2, b2):
    v_idx = hyperedge_index[0]
    e_idx = hyperedge_index[1]
    gvc = v_idx.reshape(NW, NCH_C, BC)
    gec = e_idx.reshape(NW, NCH_C, BC)
    gv2 = v_idx.reshape(NS, NCH_S, B)
    ge2 = e_idx.reshape(NS, NCH_S, B)
    gg = jnp.stack([
        jnp.stack([gv2, gv2 + MP]),
        jnp.stack([ge2, ge2 + MP]),
    ])
    ss = jnp.stack([ge2, gv2])
    b1r = b1.reshape(1, D)
    b2r = b2.reshape(1, D)

    dvp2, dep2 = _sc_counts(gvc, gec)
    dvp = dvp2.T

    table = _tc_lin1(x, W1, b1r, dvp)

    def body(t, table):
        phv = jnp.full((16,), t % 2, jnp.int32)
        mdv = jnp.full((16,), 1 - t % 2, jnp.int32)
        part = _sc_seg(table.reshape(NC * MP, H), gg, ss, dep2, phv, mdv)
        return lax.cond(t == 1,
                        lambda p: _tc_lin2(p, dvp, W2, b2r),
                        lambda p: p,
                        part)

    table = lax.fori_loop(0, 4, body, table)
    return _tc_fin(table, dvp)


# speedup vs baseline: 1.0592x; 1.0592x over previous
import functools

import jax
import jax.numpy as jnp
from jax import lax
from jax.experimental import pallas as pl
from jax.experimental.pallas import tpu as pltpu
from jax.experimental.pallas import tpu_sc as plsc

N = 10000
M = 10000
E = 320000
D = 128
H = D // 2

NC = 2
NS = 16
NW = NC * NS
B = 125
NBUF = 4
BC = 80
NCH_C = E // (NW * BC)
NCH_S = E // (NS * B)
MP = 10240
RPT = MP // NS
ZCH = RPT // 5

_MESH = dict(core_axis_name="c", subcore_axis_name="s")


def _sc_count_body(vidx_h, eidx_h, dv_out, de_out,
                   vidx_v, eidx_v, ones_v, zbuf, acc):
    c = lax.axis_index("c")
    s = lax.axis_index("s")
    wid = c * NS + s
    pltpu.sync_copy(vidx_h.at[wid], vidx_v)
    pltpu.sync_copy(eidx_h.at[wid], eidx_v)

    ones16 = jnp.ones((16,), jnp.float32)
    zero16 = jnp.zeros((16,), jnp.float32)

    @pl.loop(0, BC // 16)
    def _(i):
        ones_v[pl.ds(i * 16, 16)] = ones16

    @pl.loop(0, RPT // 16)
    def _(i):
        zbuf[pl.ds(i * 16, 16)] = zero16

    for idx_v, out in ((vidx_v, dv_out), (eidx_v, de_out)):
        pltpu.sync_copy(zbuf, acc.at[pl.ds(s * RPT, RPT)])
        plsc.subcore_barrier()

        @pl.loop(0, NCH_C)
        def _(j, idx_v=idx_v):
            pltpu.sync_copy(ones_v, acc.at[idx_v.at[j]], add=True)

        plsc.subcore_barrier()
        pltpu.sync_copy(acc.at[pl.ds(s * RPT, RPT)],
                        out.at[c, pl.ds(s * RPT, RPT)])
        plsc.subcore_barrier()


@functools.partial(
    pl.kernel,
    out_type=(jax.ShapeDtypeStruct((NC, MP), jnp.float32),
              jax.ShapeDtypeStruct((NC, MP), jnp.float32)),
    mesh=plsc.VectorSubcoreMesh(**_MESH),
    scratch_types=[
        pltpu.VMEM((NCH_C, BC), jnp.int32),
        pltpu.VMEM((NCH_C, BC), jnp.int32),
        pltpu.VMEM((BC,), jnp.float32),
        pltpu.VMEM((RPT,), jnp.float32),
        pltpu.VMEM_SHARED((MP,), jnp.float32),
    ],
    compiler_params=pltpu.CompilerParams(use_tc_tiling_on_sc=False),
)
def _sc_counts(vidx_h, eidx_h, dv_out, de_out,
               vidx_v, eidx_v, ones_v, zbuf, acc):
    _sc_count_body(vidx_h, eidx_h, dv_out, de_out,
                   vidx_v, eidx_v, ones_v, zbuf, acc)


def _sc_seg_body(table_h, gidx_h, sidx_h, deg_h, ph_h, md_h, out,
                 gidx_v, sidx_v, rows, zbuf, dg_v, sc_v, pm_v, gsems, ssems,
                 acc):
    c = lax.axis_index("c")
    s = lax.axis_index("s")
    pltpu.sync_copy(ph_h, pm_v.at[0])
    pltpu.sync_copy(md_h, pm_v.at[1])
    ph = lax.reduce_max(pm_v[0], axes=(0,))
    pltpu.sync_copy(gidx_h.at[ph, c, s], gidx_v)
    pltpu.sync_copy(sidx_h.at[ph, s], sidx_v)
    pltpu.sync_copy(deg_h.at[0, pl.ds(s * RPT, RPT)], dg_v.at[0])
    pltpu.sync_copy(deg_h.at[1, pl.ds(s * RPT, RPT)], dg_v.at[1])

    zero16 = jnp.zeros((16,), jnp.float32)
    one16 = jnp.ones((16,), jnp.float32)

    @pl.loop(0, ZCH)
    def _(i):
        for k in range(H // 16):
            zbuf[i, pl.ds(16 * k, 16)] = zero16

    md = pm_v[1]

    @pl.loop(0, RPT // 16)
    def _(i):
        d = dg_v[0, pl.ds(i * 16, 16)] + dg_v[1, pl.ds(i * 16, 16)]
        dei = jnp.where(d > 0, 1.0 / jnp.where(d > 0, d, 1.0), 0.0)
        sc_v[pl.ds(i * 16, 16)] = jnp.where(md > 0, dei, one16)

    for k in range(5):
        pltpu.sync_copy(zbuf, acc.at[pl.ds(s * RPT + k * ZCH, ZCH)])
    plsc.subcore_barrier()

    def _gather(j, b):
        return pltpu.make_async_copy(table_h.at[gidx_v.at[j]],
                                     rows.at[b], gsems[b])

    def _scatter(j, b):
        return pltpu.make_async_copy(rows.at[b], acc.at[sidx_v.at[j]],
                                     ssems[b])

    for k in range(NBUF - 1):
        _gather(k, k).start()

    @pl.loop(0, NCH_S, step=NBUF)
    def _(j):
        for b in range(NBUF):
            jj = j + b
            nxt = jj + NBUF - 1
            bn = (b + NBUF - 1) % NBUF

            def _refill(jj=jj, bn=bn, nxt=nxt):
                _scatter(jj - 1, bn).wait()

                @pl.when(nxt < NCH_S)
                def _():
                    _gather(nxt, bn).start()

            if b == 0:
                @pl.when(j > 0)
                def _():
                    _refill()
                @pl.when(j == 0)
                def _():
                    @pl.when(NBUF - 1 < NCH_S)
                    def _():
                        _gather(NBUF - 1, NBUF - 1).start()
            else:
                _refill()

            _gather(jj, b).wait()
            pltpu.async_copy(rows.at[b], acc.at[sidx_v.at[jj]],
                             ssems[b], add=True)

    _scatter(NCH_S - 1, (NCH_S - 1) % NBUF).wait()
    plsc.subcore_barrier()

    md_s = lax.reduce_max(pm_v[1], axes=(0,))

    @pl.when(md_s > 0)
    def _():
        for k in range(5):
            base = s * RPT + k * ZCH
            pltpu.sync_copy(acc.at[pl.ds(base, ZCH)], zbuf)

            @pl.loop(0, ZCH // 16)
            def _(g, k=k):
                scvec = sc_v[pl.ds(k * ZCH + g * 16, 16)]
                for rr in range(16):
                    r = g * 16 + rr
                    for kk in range(H // 16):
                        zbuf[r, pl.ds(kk * 16, 16)] = (
                            zbuf[r, pl.ds(kk * 16, 16)] * scvec[rr])

            pltpu.sync_copy(zbuf, out.at[c, pl.ds(base, ZCH)])

    @pl.when(md_s == 0)
    def _():
        pltpu.sync_copy(acc.at[pl.ds(s * RPT, RPT)],
                        out.at[c, pl.ds(s * RPT, RPT)])


@functools.partial(
    pl.kernel,
    out_type=jax.ShapeDtypeStruct((NC, MP, H), jnp.float32),
    mesh=plsc.VectorSubcoreMesh(**_MESH),
    scratch_types=[
        pltpu.VMEM((NCH_S, B), jnp.int32),
        pltpu.VMEM((NCH_S, B), jnp.int32),
        pltpu.VMEM((NBUF, B, H), jnp.float32),
        pltpu.VMEM((ZCH, H), jnp.float32),
        pltpu.VMEM((NC, RPT), jnp.float32),
        pltpu.VMEM((RPT,), jnp.float32),
        pltpu.VMEM((2, 16), jnp.int32),
        [pltpu.SemaphoreType.DMA] * NBUF,
        [pltpu.SemaphoreType.DMA] * NBUF,
        pltpu.VMEM_SHARED((MP, H), jnp.float32),
    ],
    compiler_params=pltpu.CompilerParams(use_tc_tiling_on_sc=False,
                                         needs_layout_passes=False),
)
def _sc_seg(table_h, gidx_h, sidx_h, deg_h, ph_h, md_h, out,
            gidx_v, sidx_v, rows, zbuf, dg_v, sc_v, pm_v, gsems, ssems, acc):
    _sc_seg_body(table_h, gidx_h, sidx_h, deg_h, ph_h, md_h, out,
                 gidx_v, sidx_v, rows, zbuf, dg_v, sc_v, pm_v, gsems, ssems,
                 acc)


R = 400
GRID = N // R


def _dvis_of(dvp_blk):
    dv = dvp_blk[:, 0] + dvp_blk[:, 1]
    return jnp.where(dv > 0, lax.rsqrt(jnp.where(dv > 0, dv, 1.0)), 0.0)


def _half_select(full, cid):
    return jnp.where(cid == 0, full[:, :H], full[:, H:])


def _tc_lin1_body(x_ref, w_ref, b_ref, dvp_ref, o_ref):
    cid = pl.program_id(1)
    h = lax.dot_general(x_ref[...], w_ref[...],
                        (((1,), (1,)), ((), ())),
                        preferred_element_type=jnp.float32)
    h = (h + b_ref[...]) * _dvis_of(dvp_ref[...])[:, None]
    o_ref[0] = _half_select(h, cid)


def _tc_lin1(x, W1, b1, dvp):
    return pl.pallas_call(
        _tc_lin1_body,
        grid=(GRID, NC),
        in_specs=[
            pl.BlockSpec((R, D), lambda i, c: (i, 0)),
            pl.BlockSpec((D, D), lambda i, c: (0, 0)),
            pl.BlockSpec((1, D), lambda i, c: (0, 0)),
            pl.BlockSpec((R, NC), lambda i, c: (i, 0)),
        ],
        out_specs=pl.BlockSpec((1, R, H), lambda i, c: (c, i, 0)),
        out_shape=jax.ShapeDtypeStruct((NC, MP, H), jnp.float32),
    )(x, W1, b1, dvp)


def _tc_lin2_body(xo_ref, dvp_ref, w_ref, b_ref, o_ref):
    cid = pl.program_id(1)
    dvis = _dvis_of(dvp_ref[...])
    t = jnp.concatenate([xo_ref[0], xo_ref[1]], axis=1)
    t = jax.nn.relu(t * dvis[:, None])
    h = lax.dot_general(t, w_ref[...], (((1,), (1,)), ((), ())),
                        preferred_element_type=jnp.float32)
    h = (h + b_ref[...]) * dvis[:, None]
    o_ref[0] = _half_select(h, cid)


def _tc_lin2(xo, dvp, W2, b2):
    return pl.pallas_call(
        _tc_lin2_body,
        grid=(GRID, NC),
        in_specs=[
            pl.BlockSpec((NC, R, H), lambda i, c: (0, i, 0)),
            pl.BlockSpec((R, NC), lambda i, c: (i, 0)),
            pl.BlockSpec((D, D), lambda i, c: (0, 0)),
            pl.BlockSpec((1, D), lambda i, c: (0, 0)),
        ],
        out_specs=pl.BlockSpec((1, R, H), lambda i, c: (c, i, 0)),
        out_shape=jax.ShapeDtypeStruct((NC, MP, H), jnp.float32),
    )(xo, dvp, W2, b2)


def _tc_fin_body(xo_ref, dvp_ref, o_ref):
    dvis = _dvis_of(dvp_ref[...])[:, None]
    o_ref[...] = jax.nn.relu(
        jnp.concatenate([xo_ref[0], xo_ref[1]], axis=1) * dvis)


def _tc_fin(xo, dvp):
    return pl.pallas_call(
        _tc_fin_body,
        grid=(GRID,),
        in_specs=[
            pl.BlockSpec((NC, R, H), lambda i: (0, i, 0)),
            pl.BlockSpec((R, NC), lambda i: (i, 0)),
        ],
        out_specs=pl.BlockSpec((R, D), lambda i: (i, 0)),
        out_shape=jax.ShapeDtypeStruct((N, D), jnp.float32),
    )(xo, dvp)


def kernel(x, hyperedge_index, W1, b1, W2, b2):
    v_idx = hyperedge_index[0]
    e_idx = hyperedge_index[1]
    gvc = v_idx.reshape(NW, NCH_C, BC)
    gec = e_idx.reshape(NW, NCH_C, BC)
    gv2 = v_idx.reshape(NS, NCH_S, B)
    ge2 = e_idx.reshape(NS, NCH_S, B)
    gg = jnp.stack([
        jnp.stack([gv2, gv2 + MP]),
        jnp.stack([ge2, ge2 + MP]),
    ])
    ss = jnp.stack([ge2, gv2])
    b1r = b1.reshape(1, D)
    b2r = b2.reshape(1, D)

    dvp2, dep2 = _sc_counts(gvc, gec)
    dvp = dvp2.T

    table = _tc_lin1(x, W1, b1r, dvp)

    def body(t, table):
        phv = jnp.full((16,), t % 2, jnp.int32)
        mdv = jnp.full((16,), 1 - t % 2, jnp.int32)
        part = _sc_seg(table.reshape(NC * MP, H), gg, ss, dep2, phv, mdv)
        return lax.cond(t == 1,
                        lambda p: _tc_lin2(p, dvp, W2, b2r),
                        lambda p: p,
                        part)

    table = lax.fori_loop(0, 4, body, table)
    return _tc_fin(table, dvp)


# speedup vs baseline: 1.1754x; 1.1097x over previous
import functools

import jax
import jax.numpy as jnp
from jax import lax
from jax.experimental import pallas as pl
from jax.experimental.pallas import tpu as pltpu
from jax.experimental.pallas import tpu_sc as plsc

N = 10000
M = 10000
E = 320000
D = 128
H = D // 2

NC = 2
NS = 16
NW = NC * NS
B = 125
NBUF = 4
BC = 80
NCH_C = E // (NW * BC)
NCH_S = E // (NS * B)
MP = 10240
RPT = MP // NS
ZCH = RPT // 5

_MESH = dict(core_axis_name="c", subcore_axis_name="s")


def _sc_count_body(vidx_h, eidx_h, dv_out, de_out,
                   vidx_v, eidx_v, ones_v, zbuf, acc):
    c = lax.axis_index("c")
    s = lax.axis_index("s")
    wid = c * NS + s
    pltpu.sync_copy(vidx_h.at[wid], vidx_v)
    pltpu.sync_copy(eidx_h.at[wid], eidx_v)

    ones16 = jnp.ones((16,), jnp.float32)
    zero16 = jnp.zeros((16,), jnp.float32)

    @pl.loop(0, BC // 16)
    def _(i):
        ones_v[pl.ds(i * 16, 16)] = ones16

    @pl.loop(0, RPT // 16)
    def _(i):
        zbuf[pl.ds(i * 16, 16)] = zero16

    for idx_v, out in ((vidx_v, dv_out), (eidx_v, de_out)):
        pltpu.sync_copy(zbuf, acc.at[pl.ds(s * RPT, RPT)])
        plsc.subcore_barrier()

        @pl.loop(0, NCH_C)
        def _(j, idx_v=idx_v):
            pltpu.sync_copy(ones_v, acc.at[idx_v.at[j]], add=True)

        plsc.subcore_barrier()
        pltpu.sync_copy(acc.at[pl.ds(s * RPT, RPT)],
                        out.at[c, pl.ds(s * RPT, RPT)])
        plsc.subcore_barrier()


@functools.partial(
    pl.kernel,
    out_type=(jax.ShapeDtypeStruct((NC, MP), jnp.float32),
              jax.ShapeDtypeStruct((NC, MP), jnp.float32)),
    mesh=plsc.VectorSubcoreMesh(**_MESH),
    scratch_types=[
        pltpu.VMEM((NCH_C, BC), jnp.int32),
        pltpu.VMEM((NCH_C, BC), jnp.int32),
        pltpu.VMEM((BC,), jnp.float32),
        pltpu.VMEM((RPT,), jnp.float32),
        pltpu.VMEM_SHARED((MP,), jnp.float32),
    ],
    compiler_params=pltpu.CompilerParams(use_tc_tiling_on_sc=False),
)
def _sc_counts(vidx_h, eidx_h, dv_out, de_out,
               vidx_v, eidx_v, ones_v, zbuf, acc):
    _sc_count_body(vidx_h, eidx_h, dv_out, de_out,
                   vidx_v, eidx_v, ones_v, zbuf, acc)


def _ring(table2d, gidx_v, sidx_v, rows, gsems, ssems, acc):
    def _gather(j, b):
        return pltpu.make_async_copy(table2d.at[gidx_v.at[j]],
                                     rows.at[b], gsems[b])

    def _scatter(j, b):
        return pltpu.make_async_copy(rows.at[b], acc.at[sidx_v.at[j]],
                                     ssems[b])

    for k in range(NBUF - 1):
        _gather(k, k).start()

    @pl.loop(0, NCH_S, step=NBUF)
    def _(j):
        for b in range(NBUF):
            jj = j + b
            nxt = jj + NBUF - 1
            bn = (b + NBUF - 1) % NBUF

            def _refill(jj=jj, bn=bn, nxt=nxt):
                _scatter(jj - 1, bn).wait()

                @pl.when(nxt < NCH_S)
                def _():
                    _gather(nxt, bn).start()

            if b == 0:
                @pl.when(j > 0)
                def _():
                    _refill()
                @pl.when(j == 0)
                def _():
                    _gather(NBUF - 1, NBUF - 1).start()
            else:
                _refill()

            _gather(jj, b).wait()
            pltpu.async_copy(rows.at[b], acc.at[sidx_v.at[jj]],
                             ssems[b], add=True)

    _scatter(NCH_S - 1, (NCH_S - 1) % NBUF).wait()


def _sc_segd_body(table_h, g0_h, s0_h, g1_h, s1_h, dep_h, out, tmp,
                  gidx_v, sidx_v, rows, zbuf, dg_v, sc_v, gsems, ssems, acc):
    c = lax.axis_index("c")
    s = lax.axis_index("s")
    pltpu.sync_copy(g0_h.at[c, s], gidx_v)
    pltpu.sync_copy(s0_h.at[s], sidx_v)
    pltpu.sync_copy(dep_h.at[0, pl.ds(s * RPT, RPT)], dg_v.at[0])
    pltpu.sync_copy(dep_h.at[1, pl.ds(s * RPT, RPT)], dg_v.at[1])

    zero16 = jnp.zeros((16,), jnp.float32)

    def _zero_zbuf():
        @pl.loop(0, ZCH)
        def _(i):
            for k in range(H // 16):
                zbuf[i, pl.ds(16 * k, 16)] = zero16

    def _zero_acc():
        for k in range(5):
            pltpu.sync_copy(zbuf, acc.at[pl.ds(s * RPT + k * ZCH, ZCH)])

    _zero_zbuf()

    @pl.loop(0, RPT // 16)
    def _(i):
        d = dg_v[0, pl.ds(i * 16, 16)] + dg_v[1, pl.ds(i * 16, 16)]
        sc_v[pl.ds(i * 16, 16)] = jnp.where(
            d > 0, 1.0 / jnp.where(d > 0, d, 1.0), 0.0)

    _zero_acc()
    plsc.subcore_barrier()

    _ring(table_h, gidx_v, sidx_v, rows, gsems, ssems, acc)
    plsc.subcore_barrier()

    for k in range(5):
        base = s * RPT + k * ZCH
        pltpu.sync_copy(acc.at[pl.ds(base, ZCH)], zbuf)

        @pl.loop(0, ZCH // 16)
        def _(g, k=k):
            scvec = sc_v[pl.ds(k * ZCH + g * 16, 16)]
            for rr in range(16):
                r = g * 16 + rr
                for kk in range(H // 16):
                    zbuf[r, pl.ds(kk * 16, 16)] = (
                        zbuf[r, pl.ds(kk * 16, 16)] * scvec[rr])

        pltpu.sync_copy(zbuf, tmp.at[pl.ds(c * MP + base, ZCH)])

    _zero_zbuf()
    _zero_acc()
    pltpu.sync_copy(g1_h.at[c, s], gidx_v)
    pltpu.sync_copy(s1_h.at[s], sidx_v)
    plsc.subcore_barrier()

    _ring(tmp, gidx_v, sidx_v, rows, gsems, ssems, acc)
    plsc.subcore_barrier()

    pltpu.sync_copy(acc.at[pl.ds(s * RPT, RPT)],
                    out.at[c, pl.ds(s * RPT, RPT)])


@functools.partial(
    pl.kernel,
    out_type=(jax.ShapeDtypeStruct((NC, MP, H), jnp.float32),
              jax.ShapeDtypeStruct((NC * MP, H), jnp.float32)),
    mesh=plsc.VectorSubcoreMesh(**_MESH),
    scratch_types=[
        pltpu.VMEM((NCH_S, B), jnp.int32),
        pltpu.VMEM((NCH_S, B), jnp.int32),
        pltpu.VMEM((NBUF, B, H), jnp.float32),
        pltpu.VMEM((ZCH, H), jnp.float32),
        pltpu.VMEM((NC, RPT), jnp.float32),
        pltpu.VMEM((RPT,), jnp.float32),
        [pltpu.SemaphoreType.DMA] * NBUF,
        [pltpu.SemaphoreType.DMA] * NBUF,
        pltpu.VMEM_SHARED((MP, H), jnp.float32),
    ],
    compiler_params=pltpu.CompilerParams(use_tc_tiling_on_sc=False,
                                         needs_layout_passes=False),
)
def _sc_segd(table_h, g0_h, s0_h, g1_h, s1_h, dep_h, out, tmp,
             gidx_v, sidx_v, rows, zbuf, dg_v, sc_v, gsems, ssems, acc):
    _sc_segd_body(table_h, g0_h, s0_h, g1_h, s1_h, dep_h, out, tmp,
                  gidx_v, sidx_v, rows, zbuf, dg_v, sc_v, gsems, ssems, acc)


R = 400
GRID = N // R


def _dvis_of(dvp_blk):
    dv = dvp_blk[:, 0] + dvp_blk[:, 1]
    return jnp.where(dv > 0, lax.rsqrt(jnp.where(dv > 0, dv, 1.0)), 0.0)


def _half_select(full, cid):
    return jnp.where(cid == 0, full[:, :H], full[:, H:])


def _tc_lin1_body(x_ref, w_ref, b_ref, dvp_ref, o_ref):
    cid = pl.program_id(1)
    h = lax.dot_general(x_ref[...], w_ref[...],
                        (((1,), (1,)), ((), ())),
                        preferred_element_type=jnp.float32)
    h = (h + b_ref[...]) * _dvis_of(dvp_ref[...])[:, None]
    o_ref[0] = _half_select(h, cid)


def _tc_lin1(x, W1, b1, dvp):
    return pl.pallas_call(
        _tc_lin1_body,
        grid=(GRID, NC),
        in_specs=[
            pl.BlockSpec((R, D), lambda i, c: (i, 0)),
            pl.BlockSpec((D, D), lambda i, c: (0, 0)),
            pl.BlockSpec((1, D), lambda i, c: (0, 0)),
            pl.BlockSpec((R, NC), lambda i, c: (i, 0)),
        ],
        out_specs=pl.BlockSpec((1, R, H), lambda i, c: (c, i, 0)),
        out_shape=jax.ShapeDtypeStruct((NC, MP, H), jnp.float32),
    )(x, W1, b1, dvp)


def _tc_lin2_body(xo_ref, dvp_ref, w_ref, b_ref, o_ref):
    cid = pl.program_id(1)
    dvis = _dvis_of(dvp_ref[...])
    t = jnp.concatenate([xo_ref[0], xo_ref[1]], axis=1)
    t = jax.nn.relu(t * dvis[:, None])
    h = lax.dot_general(t, w_ref[...], (((1,), (1,)), ((), ())),
                        preferred_element_type=jnp.float32)
    h = (h + b_ref[...]) * dvis[:, None]
    o_ref[0] = _half_select(h, cid)


def _tc_lin2(xo, dvp, W2, b2):
    return pl.pallas_call(
        _tc_lin2_body,
        grid=(GRID, NC),
        in_specs=[
            pl.BlockSpec((NC, R, H), lambda i, c: (0, i, 0)),
            pl.BlockSpec((R, NC), lambda i, c: (i, 0)),
            pl.BlockSpec((D, D), lambda i, c: (0, 0)),
            pl.BlockSpec((1, D), lambda i, c: (0, 0)),
        ],
        out_specs=pl.BlockSpec((1, R, H), lambda i, c: (c, i, 0)),
        out_shape=jax.ShapeDtypeStruct((NC, MP, H), jnp.float32),
    )(xo, dvp, W2, b2)


def _tc_fin_body(xo_ref, dvp_ref, o_ref):
    dvis = _dvis_of(dvp_ref[...])[:, None]
    o_ref[...] = jax.nn.relu(
        jnp.concatenate([xo_ref[0], xo_ref[1]], axis=1) * dvis)


def _tc_fin(xo, dvp):
    return pl.pallas_call(
        _tc_fin_body,
        grid=(GRID,),
        in_specs=[
            pl.BlockSpec((NC, R, H), lambda i: (0, i, 0)),
            pl.BlockSpec((R, NC), lambda i: (i, 0)),
        ],
        out_specs=pl.BlockSpec((R, D), lambda i: (i, 0)),
        out_shape=jax.ShapeDtypeStruct((N, D), jnp.float32),
    )(xo, dvp)


def kernel(x, hyperedge_index, W1, b1, W2, b2):
    v_idx = hyperedge_index[0]
    e_idx = hyperedge_index[1]
    gvc = v_idx.reshape(NW, NCH_C, BC)
    gec = e_idx.reshape(NW, NCH_C, BC)
    gv2 = v_idx.reshape(NS, NCH_S, B)
    ge2 = e_idx.reshape(NS, NCH_S, B)
    gg0 = jnp.stack([gv2, gv2 + MP])
    gg1 = jnp.stack([ge2, ge2 + MP])
    b1r = b1.reshape(1, D)
    b2r = b2.reshape(1, D)

    dvp2, dep2 = _sc_counts(gvc, gec)
    dvp = dvp2.T

    table = _tc_lin1(x, W1, b1r, dvp)

    def body(layer, table):
        part, _ = _sc_segd(table.reshape(NC * MP, H),
                           gg0, ge2, gg1, gv2, dep2)
        return lax.cond(layer == 0,
                        lambda p: _tc_lin2(p, dvp, W2, b2r),
                        lambda p: p,
                        part)

    table = lax.fori_loop(0, 2, body, table)
    return _tc_fin(table, dvp)
